# baseline (speedup 1.0000x reference)
import functools

import jax
import jax.numpy as jnp
from jax import lax
from jax.experimental import pallas as pl
from jax.experimental.pallas import tpu as pltpu
import jax.experimental.pallas.tpu_sc as plsc

B, P, C_IN = 4, 4096, 128
S, K = 1024, 64
HID, C_OUT = 256, 256
R2 = 0.2 * 0.2
TS = 128
G = 8

_INTERPRET = False


def _fps_body(pos_ref, sel_ref):
    px = pos_ref[:, 0, :]
    py = pos_ref[:, 1, :]
    pz = pos_ref[:, 2, :]
    lane = jax.lax.broadcasted_iota(jnp.int32, (B, P), 1).astype(jnp.float32)

    c0x = px[:, 0:1]
    c0y = py[:, 0:1]
    c0z = pz[:, 0:1]
    dx = px - c0x
    dy = py - c0y
    dz = pz - c0z
    mind = dx * dx + dy * dy + dz * dz
    lane_s = jax.lax.broadcasted_iota(jnp.int32, (3 * B, S), 1)
    acc = jnp.where(lane_s == 0, jnp.concatenate([c0x, c0y, c0z], axis=0),
                    0.0)

    def body(s, carry):
        mind, acc = carry
        m = jnp.max(mind, axis=1, keepdims=True)
        nxt = jnp.min(jnp.where(mind == m, lane, float(P)), axis=1,
                      keepdims=True)
        oh = lane == nxt
        cx = jnp.sum(jnp.where(oh, px, 0.0), axis=1, keepdims=True)
        cy = jnp.sum(jnp.where(oh, py, 0.0), axis=1, keepdims=True)
        cz = jnp.sum(jnp.where(oh, pz, 0.0), axis=1, keepdims=True)
        ddx = px - cx
        ddy = py - cy
        ddz = pz - cz
        d = ddx * ddx + ddy * ddy + ddz * ddz
        acc = jnp.where(lane_s == s, jnp.concatenate([cx, cy, cz], axis=0),
                        acc)
        return jnp.minimum(mind, d), acc

    _, acc = jax.lax.fori_loop(1, S, body, (mind, acc))
    sel_ref[:, :] = acc


def _run_fps(pos_t):
    return pl.pallas_call(
        _fps_body,
        out_shape=jax.ShapeDtypeStruct((3 * B, S), jnp.float32),
        interpret=_INTERPRET,
    )(pos_t)


def _topk_body(pos_ref, sel_ref, nbr_ref):
    px = pos_ref[0, 0, :][None, :]
    py = pos_ref[0, 1, :][None, :]
    pz = pos_ref[0, 2, :][None, :]
    sx = sel_ref[0, :, 0:1]
    sy = sel_ref[0, :, 1:2]
    sz = sel_ref[0, :, 2:3]
    ss = sx * sx + sy * sy + sz * sz
    pp = px * px + py * py + pz * pz
    bxl = lambda v: v.astype(jnp.bfloat16).astype(jnp.float32)
    dot = (bxl(sx) * bxl(px) + bxl(sy) * bxl(py) + bxl(sz) * bxl(pz))
    d = jnp.maximum(ss + pp - 2.0 * dot, 0.0)

    lane = jax.lax.broadcasted_iota(jnp.int32, (TS, P), 1).astype(jnp.float32)
    inf = jnp.float32(jnp.inf)
    FW = 128
    NF = P // FW
    lanes_c = [lane[:, c * FW:(c + 1) * FW] for c in range(NF)]
    ik0 = None
    for k in range(K):
        v = d[:, 0:FW]
        lw = lanes_c[0]
        for c in range(1, NF):
            dc = d[:, c * FW:(c + 1) * FW]
            mask = dc < v
            lw = jnp.where(mask, lanes_c[c], lw)
            v = jnp.minimum(dc, v)
        m = jnp.min(v, axis=1, keepdims=True)
        ik = jnp.min(jnp.where(v == m, lw, float(P)), axis=1,
                     keepdims=True)
        if k == 0:
            ik0 = ik
            nbr_ref[0, :, 0:1] = ik.astype(jnp.int32)
        else:
            sel = jnp.where(m <= R2, ik, ik0)
            nbr_ref[0, :, k:k + 1] = sel.astype(jnp.int32)
        d = jnp.where(lane == ik, inf, d)


def _run_topk(pos_t, sel_t):
    return pl.pallas_call(
        _topk_body,
        grid=(B, S // TS),
        in_specs=[
            pl.BlockSpec((1, 3, P), lambda b, t: (b, 0, 0)),
            pl.BlockSpec((1, TS, 3), lambda b, t: (b, t, 0)),
        ],
        out_specs=pl.BlockSpec((1, TS, K), lambda b, t: (b, t, 0)),
        out_shape=jax.ShapeDtypeStruct((B, S, K), jnp.int32),
        interpret=_INTERPRET,
    )(pos_t, sel_t)


D_PAD = 256
ROWS = B * S * K
NW = 32
RPW = ROWS // NW
CH = 128
NCH = RPW // CH


def _run_gather_sc(table_pad, idx_resh):
    mesh = plsc.VectorSubcoreMesh(core_axis_name="c", subcore_axis_name="s")

    @functools.partial(
        pl.kernel, mesh=mesh,
        out_type=jax.ShapeDtypeStruct((ROWS, D_PAD), jnp.float32),
        scratch_types=[
            pltpu.VMEM((NCH, CH), jnp.int32),
            pltpu.VMEM((CH, D_PAD), jnp.float32),
            pltpu.SemaphoreType.DMA,
        ],
    )
    def k(table_hbm, idx_hbm, out_hbm, idx_v, rows_v, sem):
        wid = lax.axis_index("s") * 2 + lax.axis_index("c")
        base = wid * RPW
        pltpu.sync_copy(idx_hbm.at[wid], idx_v)

        def chunk(i, carry):
            pltpu.async_copy(table_hbm.at[idx_v.at[i]], rows_v, sem).wait()
            pltpu.sync_copy(rows_v, out_hbm.at[pl.ds(base + i * CH, CH)])
            return carry

        lax.fori_loop(0, NCH, chunk, 0)

    return k(table_pad, idx_resh)


def _conv_body(nbr_ref, tab_ref, sel_ref, w1_ref, b1_ref, w2_ref, b2_ref,
               out_ref, feat):
    for r in range(G):
        for k in range(K):
            j = nbr_ref[r, k]
            feat[pl.ds(r * K + k, 1), :] = tab_ref[0, pl.ds(j, 1), :]
    for r in range(G):
        sel_row = sel_ref[0, r:r + 1, :]
        blk = feat[pl.ds(r * K, K), C_IN:C_IN + 3]
        feat[pl.ds(r * K, K), C_IN:C_IN + 3] = blk - sel_row
    h = jnp.dot(feat[:, :], w1_ref[:, :],
                preferred_element_type=jnp.float32) + b1_ref[:, :]
    h = jnp.maximum(h, 0.0)
    g = jnp.dot(h, w2_ref[:, :],
                preferred_element_type=jnp.float32) + b2_ref[:, :]
    g = jnp.maximum(g, 0.0)
    out_ref[0, :, :] = jnp.max(g.reshape(G, K, C_OUT), axis=1)


def _run_conv(nbr_flat, table, sel_t, W1, b1, W2, b2):
    nblocks = S // G
    return pl.pallas_call(
        _conv_body,
        grid=(B, nblocks),
        in_specs=[
            pl.BlockSpec((G, K), lambda b, c: (b * nblocks + c, 0),
                         memory_space=pltpu.SMEM),
            pl.BlockSpec((1, P, C_IN + 3), lambda b, c: (b, 0, 0)),
            pl.BlockSpec((1, G, 3), lambda b, c: (b, c, 0)),
            pl.BlockSpec((C_IN + 3, HID), lambda b, c: (0, 0)),
            pl.BlockSpec((1, HID), lambda b, c: (0, 0)),
            pl.BlockSpec((HID, C_OUT), lambda b, c: (0, 0)),
            pl.BlockSpec((1, C_OUT), lambda b, c: (0, 0)),
        ],
        out_specs=pl.BlockSpec((1, G, C_OUT), lambda b, c: (b * nblocks + c, 0, 0)),
        out_shape=jax.ShapeDtypeStruct((B * S // G, G, C_OUT), jnp.float32),
        scratch_shapes=[pltpu.VMEM((G * K, C_IN + 3), jnp.float32)],
        interpret=_INTERPRET,
    )(nbr_flat, table, sel_t, W1, b1, W2, b2)


GD = 32


def _dconv_body(gath_ref, sel_ref, w1_ref, b1_ref, w2_ref, b2_ref,
                out_ref, feat):
    feat[:, :] = gath_ref[:, :]
    for r in range(GD):
        sel_row = sel_ref[r:r + 1, :]
        blk = feat[pl.ds(r * K, K), C_IN:C_IN + 3]
        feat[pl.ds(r * K, K), C_IN:C_IN + 3] = blk - sel_row
    h = jnp.dot(feat[:, :], w1_ref[:, :],
                preferred_element_type=jnp.float32) + b1_ref[:, :]
    h = jnp.maximum(h, 0.0)
    g = jnp.dot(h, w2_ref[:, :],
                preferred_element_type=jnp.float32) + b2_ref[:, :]
    g = jnp.maximum(g, 0.0)
    out_ref[0, :, :] = jnp.max(g.reshape(GD, K, C_OUT), axis=1)


def _run_dense_conv(gathered, sel_flat, W1p, b1, W2, b2):
    nch = B * S // GD
    return pl.pallas_call(
        _dconv_body,
        grid=(nch,),
        in_specs=[
            pl.BlockSpec((GD * K, D_PAD), lambda c: (c, 0)),
            pl.BlockSpec((GD, 3), lambda c: (c, 0)),
            pl.BlockSpec((D_PAD, HID), lambda c: (0, 0)),
            pl.BlockSpec((1, HID), lambda c: (0, 0)),
            pl.BlockSpec((HID, C_OUT), lambda c: (0, 0)),
            pl.BlockSpec((1, C_OUT), lambda c: (0, 0)),
        ],
        out_specs=pl.BlockSpec((1, GD, C_OUT), lambda c: (c, 0, 0)),
        out_shape=jax.ShapeDtypeStruct((nch, GD, C_OUT), jnp.float32),
        scratch_shapes=[pltpu.VMEM((GD * K, D_PAD), jnp.float32)],
        interpret=_INTERPRET,
    )(gathered, sel_flat, W1p, b1, W2, b2)


def kernel(x, pos, batch, W1, b1, W2, b2):
    pos_r = pos.reshape(B, P, 3)
    pos_t = pos_r.transpose(0, 2, 1)
    sel = _run_fps(pos_t)
    sel_t = sel.reshape(3, B, S).transpose(1, 2, 0)
    nbr = _run_topk(pos_t, sel_t)
    table_pad = jnp.concatenate(
        [x, pos, jnp.zeros((B * P, D_PAD - C_IN - 3), jnp.float32)], axis=1)
    idx_resh = (nbr + jnp.arange(B, dtype=jnp.int32)[:, None, None] * P
                ).reshape(NW, NCH, CH)
    gathered = _run_gather_sc(table_pad, idx_resh)
    W1p = jnp.concatenate(
        [W1, jnp.zeros((D_PAD - C_IN - 3, HID), jnp.float32)], axis=0)
    out = _run_dense_conv(gathered, sel_t.reshape(B * S, 3), W1p,
                          b1.reshape(1, HID), W2, b2.reshape(1, C_OUT))
    out = out.reshape(B * S, C_OUT)
    sel_pos = sel_t.reshape(B * S, 3)
    sel_batch = jnp.repeat(jnp.arange(B, dtype=batch.dtype), S)
    return out, sel_pos, sel_batch


# speedup vs baseline: 1.0981x; 1.0981x over previous
import functools

import jax
import jax.numpy as jnp
from jax import lax
from jax.experimental import pallas as pl
from jax.experimental.pallas import tpu as pltpu
import jax.experimental.pallas.tpu_sc as plsc

B, P, C_IN = 4, 4096, 128
S, K = 1024, 64
HID, C_OUT = 256, 256
R2 = 0.2 * 0.2
TS = 128
G = 8

_INTERPRET = False


def _fps_body(pos_ref, sel_ref):
    px = pos_ref[:, 0, :]
    py = pos_ref[:, 1, :]
    pz = pos_ref[:, 2, :]
    lane = jax.lax.broadcasted_iota(jnp.int32, (B, P), 1).astype(jnp.float32)

    c0x = px[:, 0:1]
    c0y = py[:, 0:1]
    c0z = pz[:, 0:1]
    dx = px - c0x
    dy = py - c0y
    dz = pz - c0z
    mind = dx * dx + dy * dy + dz * dz
    lane_s = jax.lax.broadcasted_iota(jnp.int32, (3 * B, S), 1)
    acc = jnp.where(lane_s == 0, jnp.concatenate([c0x, c0y, c0z], axis=0),
                    0.0)

    def body(s, carry):
        mind, acc = carry
        m = jnp.max(mind, axis=1, keepdims=True)
        nxt = jnp.min(jnp.where(mind == m, lane, float(P)), axis=1,
                      keepdims=True)
        oh = lane == nxt
        cx = jnp.sum(jnp.where(oh, px, 0.0), axis=1, keepdims=True)
        cy = jnp.sum(jnp.where(oh, py, 0.0), axis=1, keepdims=True)
        cz = jnp.sum(jnp.where(oh, pz, 0.0), axis=1, keepdims=True)
        ddx = px - cx
        ddy = py - cy
        ddz = pz - cz
        d = ddx * ddx + ddy * ddy + ddz * ddz
        acc = jnp.where(lane_s == s, jnp.concatenate([cx, cy, cz], axis=0),
                        acc)
        return jnp.minimum(mind, d), acc

    _, acc = jax.lax.fori_loop(1, S, body, (mind, acc))
    sel_ref[:, :] = acc


def _run_fps(pos_t):
    return pl.pallas_call(
        _fps_body,
        out_shape=jax.ShapeDtypeStruct((3 * B, S), jnp.float32),
        interpret=_INTERPRET,
    )(pos_t)


def _topk_body(pos_ref, sel_ref, nbr_ref):
    px = pos_ref[0, 0, :][None, :]
    py = pos_ref[0, 1, :][None, :]
    pz = pos_ref[0, 2, :][None, :]
    sx = sel_ref[0, :, 0:1]
    sy = sel_ref[0, :, 1:2]
    sz = sel_ref[0, :, 2:3]
    ss = sx * sx + sy * sy + sz * sz
    pp = px * px + py * py + pz * pz
    bxl = lambda v: v.astype(jnp.bfloat16).astype(jnp.float32)
    dot = (bxl(sx) * bxl(px) + bxl(sy) * bxl(py) + bxl(sz) * bxl(pz))
    d = jnp.maximum(ss + pp - 2.0 * dot, 0.0)

    lane = jax.lax.broadcasted_iota(jnp.int32, (TS, P), 1).astype(jnp.float32)
    inf = jnp.float32(jnp.inf)
    ik0 = None
    for k in range(K):
        m = jnp.min(d, axis=1, keepdims=True)
        ik = jnp.min(jnp.where(d == m, lane, float(P)), axis=1,
                     keepdims=True)
        if k == 0:
            ik0 = ik
            nbr_ref[0, :, 0:1] = ik.astype(jnp.int32)
        else:
            sel = jnp.where(m <= R2, ik, ik0)
            nbr_ref[0, :, k:k + 1] = sel.astype(jnp.int32)
        d = jnp.where(lane == ik, inf, d)


def _run_topk(pos_t, sel_t):
    return pl.pallas_call(
        _topk_body,
        grid=(B, S // TS),
        in_specs=[
            pl.BlockSpec((1, 3, P), lambda b, t: (b, 0, 0)),
            pl.BlockSpec((1, TS, 3), lambda b, t: (b, t, 0)),
        ],
        out_specs=pl.BlockSpec((1, TS, K), lambda b, t: (b, t, 0)),
        out_shape=jax.ShapeDtypeStruct((B, S, K), jnp.int32),
        interpret=_INTERPRET,
    )(pos_t, sel_t)


D_PAD = 256
ROWS = B * S * K
NW = 32
RPW = ROWS // NW
CH = 128
NCH = RPW // CH


def _run_gather_sc(table_pad, idx_resh):
    mesh = plsc.VectorSubcoreMesh(core_axis_name="c", subcore_axis_name="s")

    @functools.partial(
        pl.kernel, mesh=mesh,
        out_type=jax.ShapeDtypeStruct((ROWS, D_PAD), jnp.float32),
        scratch_types=[
            pltpu.VMEM((NCH, CH), jnp.int32),
            pltpu.VMEM((CH, D_PAD), jnp.float32),
            pltpu.VMEM((CH, D_PAD), jnp.float32),
            pltpu.SemaphoreType.DMA,
            pltpu.SemaphoreType.DMA,
        ],
    )
    def k(table_hbm, idx_hbm, out_hbm, idx_v, rows0, rows1, sem0, sem1):
        wid = lax.axis_index("s") * 2 + lax.axis_index("c")
        base = wid * RPW
        pltpu.sync_copy(idx_hbm.at[wid], idx_v)
        bufs = (rows0, rows1)
        sems = (sem0, sem1)
        pltpu.make_async_copy(table_hbm.at[idx_v.at[0]], rows0, sem0).start()

        def pair(p, carry):
            i = p * 2
            for par in range(2):
                buf, sem = bufs[par], sems[par]
                nbuf, nsem = bufs[1 - par], sems[1 - par]
                nxt = i + par + 1

                @pl.when(nxt < NCH)
                def _():
                    pltpu.make_async_copy(
                        table_hbm.at[idx_v.at[nxt]], nbuf, nsem).start()

                pltpu.make_async_copy(
                    table_hbm.at[idx_v.at[i + par]], buf, sem).wait()
                pltpu.sync_copy(
                    buf, out_hbm.at[pl.ds(base + (i + par) * CH, CH)])
            return carry

        lax.fori_loop(0, NCH // 2, pair, 0)

    return k(table_pad, idx_resh)


def _conv_body(nbr_ref, tab_ref, sel_ref, w1_ref, b1_ref, w2_ref, b2_ref,
               out_ref, feat):
    for r in range(G):
        for k in range(K):
            j = nbr_ref[r, k]
            feat[pl.ds(r * K + k, 1), :] = tab_ref[0, pl.ds(j, 1), :]
    for r in range(G):
        sel_row = sel_ref[0, r:r + 1, :]
        blk = feat[pl.ds(r * K, K), C_IN:C_IN + 3]
        feat[pl.ds(r * K, K), C_IN:C_IN + 3] = blk - sel_row
    h = jnp.dot(feat[:, :], w1_ref[:, :],
                preferred_element_type=jnp.float32) + b1_ref[:, :]
    h = jnp.maximum(h, 0.0)
    g = jnp.dot(h, w2_ref[:, :],
                preferred_element_type=jnp.float32) + b2_ref[:, :]
    g = jnp.maximum(g, 0.0)
    out_ref[0, :, :] = jnp.max(g.reshape(G, K, C_OUT), axis=1)


def _run_conv(nbr_flat, table, sel_t, W1, b1, W2, b2):
    nblocks = S // G
    return pl.pallas_call(
        _conv_body,
        grid=(B, nblocks),
        in_specs=[
            pl.BlockSpec((G, K), lambda b, c: (b * nblocks + c, 0),
                         memory_space=pltpu.SMEM),
            pl.BlockSpec((1, P, C_IN + 3), lambda b, c: (b, 0, 0)),
            pl.BlockSpec((1, G, 3), lambda b, c: (b, c, 0)),
            pl.BlockSpec((C_IN + 3, HID), lambda b, c: (0, 0)),
            pl.BlockSpec((1, HID), lambda b, c: (0, 0)),
            pl.BlockSpec((HID, C_OUT), lambda b, c: (0, 0)),
            pl.BlockSpec((1, C_OUT), lambda b, c: (0, 0)),
        ],
        out_specs=pl.BlockSpec((1, G, C_OUT), lambda b, c: (b * nblocks + c, 0, 0)),
        out_shape=jax.ShapeDtypeStruct((B * S // G, G, C_OUT), jnp.float32),
        scratch_shapes=[pltpu.VMEM((G * K, C_IN + 3), jnp.float32)],
        interpret=_INTERPRET,
    )(nbr_flat, table, sel_t, W1, b1, W2, b2)


GD = 32


def _dconv_body(gath_ref, sel_ref, w1_ref, b1_ref, w2_ref, b2_ref,
                out_ref, feat):
    feat[:, :] = gath_ref[:, :]
    for r in range(GD):
        sel_row = sel_ref[r:r + 1, :]
        blk = feat[pl.ds(r * K, K), C_IN:C_IN + 3]
        feat[pl.ds(r * K, K), C_IN:C_IN + 3] = blk - sel_row
    h = jnp.dot(feat[:, :], w1_ref[:, :],
                preferred_element_type=jnp.float32) + b1_ref[:, :]
    h = jnp.maximum(h, 0.0)
    g = jnp.dot(h, w2_ref[:, :],
                preferred_element_type=jnp.float32) + b2_ref[:, :]
    g = jnp.maximum(g, 0.0)
    out_ref[0, :, :] = jnp.max(g.reshape(GD, K, C_OUT), axis=1)


def _run_dense_conv(gathered, sel_flat, W1p, b1, W2, b2):
    nch = B * S // GD
    return pl.pallas_call(
        _dconv_body,
        grid=(nch,),
        in_specs=[
            pl.BlockSpec((GD * K, D_PAD), lambda c: (c, 0)),
            pl.BlockSpec((GD, 3), lambda c: (c, 0)),
            pl.BlockSpec((D_PAD, HID), lambda c: (0, 0)),
            pl.BlockSpec((1, HID), lambda c: (0, 0)),
            pl.BlockSpec((HID, C_OUT), lambda c: (0, 0)),
            pl.BlockSpec((1, C_OUT), lambda c: (0, 0)),
        ],
        out_specs=pl.BlockSpec((1, GD, C_OUT), lambda c: (c, 0, 0)),
        out_shape=jax.ShapeDtypeStruct((nch, GD, C_OUT), jnp.float32),
        scratch_shapes=[pltpu.VMEM((GD * K, D_PAD), jnp.float32)],
        interpret=_INTERPRET,
    )(gathered, sel_flat, W1p, b1, W2, b2)


def kernel(x, pos, batch, W1, b1, W2, b2):
    pos_r = pos.reshape(B, P, 3)
    pos_t = pos_r.transpose(0, 2, 1)
    sel = _run_fps(pos_t)
    sel_t = sel.reshape(3, B, S).transpose(1, 2, 0)
    nbr = _run_topk(pos_t, sel_t)
    table_pad = jnp.concatenate(
        [x, pos, jnp.zeros((B * P, D_PAD - C_IN - 3), jnp.float32)], axis=1)
    idx_resh = (nbr + jnp.arange(B, dtype=jnp.int32)[:, None, None] * P
                ).reshape(NW, NCH, CH)
    gathered = _run_gather_sc(table_pad, idx_resh)
    W1p = jnp.concatenate(
        [W1, jnp.zeros((D_PAD - C_IN - 3, HID), jnp.float32)], axis=0)
    out = _run_dense_conv(gathered, sel_t.reshape(B * S, 3), W1p,
                          b1.reshape(1, HID), W2, b2.reshape(1, C_OUT))
    out = out.reshape(B * S, C_OUT)
    sel_pos = sel_t.reshape(B * S, 3)
    sel_batch = jnp.repeat(jnp.arange(B, dtype=batch.dtype), S)
    return out, sel_pos, sel_batch
